# baseline (speedup 1.0000x reference)
import jax
import jax.numpy as jnp
from jax import lax
from jax.experimental import pallas as pl
from jax.experimental.pallas import tpu as pltpu
from jax.experimental.pallas import tpu_sc as plsc

NCLS = 19
_EPS = 1e-07


_R = 128


def _tc_body(pr_ref, gt_ref, out_ref):
    def amax(x):
        m = x[0]
        idx = jnp.zeros(m.shape, jnp.int32)
        for k in range(1, NCLS):
            xk = x[k]
            b = xk > m
            m = jnp.where(b, xk, m)
            idx = jnp.where(b, k, idx)
        return idx

    out_ref[...] = amax(gt_ref[0]) * NCLS + amax(pr_ref[0])


def _argmax_flat(y_pr, y_gt, row0, nrows):
    n, k, h, w = y_pr.shape
    nrb = nrows // _R
    rb0 = row0 // _R
    return pl.pallas_call(
        _tc_body,
        grid=(n, nrb),
        in_specs=[
            pl.BlockSpec((1, k, _R, w), lambda i, j: (i, 0, j + rb0, 0)),
            pl.BlockSpec((1, k, _R, w), lambda i, j: (i, 0, j + rb0, 0)),
        ],
        out_specs=pl.BlockSpec((_R, w), lambda i, j: (i * nrb + j, 0)),
        out_shape=jax.ShapeDtypeStruct((n * nrows, w), jnp.int32),
    )(y_pr, y_gt)


_W = 512
_NCORE = 2
_NWORK = 16 * _NCORE
_TSTRIDE = 368
_NT16 = _TSTRIDE // 16


def _make_sc_body(rows):
    rpw = rows // _NWORK
    crows = min(32, rpw)
    nch = rpw // crows

    def _sc_body(flat_hbm, out_hbm, buf0, buf1, tab, hist, sem0, sem1):
        wid = lax.axis_index("s") * _NCORE + lax.axis_index("c")
        iota = lax.iota(jnp.int32, 16)
        ones = jnp.ones((16,), jnp.float32)
        zeros = jnp.zeros((16,), jnp.float32)
        laneoff = iota * _TSTRIDE

        def _zero(i, c):
            tab[pl.ds(i * 16, 16)] = zeros
            return c

        lax.fori_loop(0, 16 * _NT16, _zero, 0)

        bufs = (buf0, buf1)
        sems = (sem0, sem1)
        nvec_row = _W // 16

        def _start(ci):
            r0 = wid * rpw + ci * crows
            return pltpu.async_copy(flat_hbm.at[pl.ds(r0, crows)],
                                    bufs[ci % 2], sems[ci % 2])

        cp = _start(0)
        for ci in range(nch):
            cp.wait()
            if ci + 1 < nch:
                cp = _start(ci + 1)
            buf = bufs[ci % 2]

            def _rowloop(r, carry, buf=buf):
                def _scat(c, cc):
                    v = buf[r, pl.ds(c * 16, 16)]
                    plsc.addupdate_scatter(tab, [laneoff + v], ones)
                    return cc

                return lax.fori_loop(0, nvec_row, _scat, carry, unroll=4)

            lax.fori_loop(0, crows, _rowloop, 0)

        def _red(j, c):
            acc = tab[pl.ds(j * 16, 16)]
            for l in range(1, 16):
                acc = acc + tab[pl.ds(l * _TSTRIDE + j * 16, 16)]
            hist[pl.ds(j * 16, 16)] = acc
            return c

        lax.fori_loop(0, _NT16, _red, 0)
        pltpu.sync_copy(hist, out_hbm.at[wid])

    return _sc_body, crows


def _sc_hist(flat2d):
    rows = flat2d.shape[0]
    body, crows = _make_sc_body(rows)
    mesh = plsc.VectorSubcoreMesh(
        core_axis_name="c", subcore_axis_name="s", num_cores=_NCORE)
    return pl.kernel(
        body,
        out_type=jax.ShapeDtypeStruct((_NWORK, _TSTRIDE), jnp.float32),
        mesh=mesh,
        compiler_params=pltpu.CompilerParams(needs_layout_passes=False),
        scratch_types=[
            pltpu.VMEM((crows, _W), jnp.int32),
            pltpu.VMEM((crows, _W), jnp.int32),
            pltpu.VMEM((16 * _TSTRIDE,), jnp.float32),
            pltpu.VMEM((_TSTRIDE,), jnp.float32),
            pltpu.SemaphoreType.DMA,
            pltpu.SemaphoreType.DMA,
        ],
    )(flat2d)


def _iou_body(h_ref, out_ref):
    conf = jnp.sum(h_ref[...], axis=0, keepdims=True)
    b = lax.broadcasted_iota(jnp.int32, (_TSTRIDE, NCLS), 0)
    i = lax.broadcasted_iota(jnp.int32, (_TSTRIDE, NCLS), 1)
    valid = b < NCLS * NCLS
    mrow = ((b // NCLS == i) & valid).astype(jnp.float32)
    mcol = ((b % NCLS == i) & valid).astype(jnp.float32)
    mtp = (b == i * (NCLS + 1)).astype(jnp.float32)
    dn = (((1,), (0,)), ((), ()))
    hp = lax.Precision.HIGHEST
    row = lax.dot_general(conf, mrow, dn, precision=hp,
                          preferred_element_type=jnp.float32)
    col = lax.dot_general(conf, mcol, dn, precision=hp,
                          preferred_element_type=jnp.float32)
    tp = lax.dot_general(conf, mtp, dn, precision=hp,
                         preferred_element_type=jnp.float32)
    iou = (tp + _EPS) / (row + col - tp + _EPS)
    mask = (row > 0).astype(jnp.float32)
    num = jnp.sum(iou * mask, keepdims=True)
    den = jnp.maximum(jnp.sum(mask, keepdims=True), 1.0)
    out_ref[...] = num / den


def _iou_tc(hists):
    return pl.pallas_call(
        _iou_body,
        out_shape=jax.ShapeDtypeStruct((1, 1), jnp.float32),
    )(hists)


_NSLICE = 4


def kernel(y_pr, y_gt):
    h = y_pr.shape[2]
    srows = h // _NSLICE
    parts = []
    for s in range(_NSLICE):
        fl = _argmax_flat(y_pr, y_gt, s * srows, srows)
        parts.append(_sc_hist(fl))
    hists = jnp.concatenate(parts, axis=0)
    out = _iou_tc(hists)
    return out[0, 0]


# speedup vs baseline: 1.0014x; 1.0014x over previous
import jax
import jax.numpy as jnp
from jax import lax
from jax.experimental import pallas as pl
from jax.experimental.pallas import tpu as pltpu
from jax.experimental.pallas import tpu_sc as plsc

NCLS = 19
_EPS = 1e-07


_R = 128


def _tc_body(pr_ref, gt_ref, out_ref):
    def amax(x):
        m = x[0]
        idx = jnp.zeros(m.shape, jnp.int32)
        for k in range(1, NCLS):
            xk = x[k]
            b = xk > m
            m = jnp.where(b, xk, m)
            idx = jnp.where(b, k, idx)
        return idx

    out_ref[...] = amax(gt_ref[0]) * NCLS + amax(pr_ref[0])


def _argmax_flat(y_pr, y_gt, row0, nrows):
    n, k, h, w = y_pr.shape
    nrb = nrows // _R
    rb0 = row0 // _R
    return pl.pallas_call(
        _tc_body,
        grid=(n, nrb),
        in_specs=[
            pl.BlockSpec((1, k, _R, w), lambda i, j: (i, 0, j + rb0, 0)),
            pl.BlockSpec((1, k, _R, w), lambda i, j: (i, 0, j + rb0, 0)),
        ],
        out_specs=pl.BlockSpec((_R, w), lambda i, j: (i * nrb + j, 0)),
        out_shape=jax.ShapeDtypeStruct((n * nrows, w), jnp.int32),
    )(y_pr, y_gt)


_W = 512
_NCORE = 2
_NWORK = 16 * _NCORE
_TSTRIDE = 368
_NT16 = _TSTRIDE // 16


def _make_sc_body(rows):
    rpw = rows // _NWORK
    nch = max(1, (rpw + 31) // 32)
    crows = rpw // nch

    def _sc_body(flat_hbm, out_hbm, buf0, buf1, tab, hist, sem0, sem1):
        wid = lax.axis_index("s") * _NCORE + lax.axis_index("c")
        iota = lax.iota(jnp.int32, 16)
        ones = jnp.ones((16,), jnp.float32)
        zeros = jnp.zeros((16,), jnp.float32)
        laneoff = iota * _TSTRIDE

        def _zero(i, c):
            tab[pl.ds(i * 16, 16)] = zeros
            return c

        lax.fori_loop(0, 16 * _NT16, _zero, 0)

        bufs = (buf0, buf1)
        sems = (sem0, sem1)
        nvec_row = _W // 16

        def _start(ci):
            r0 = wid * rpw + ci * crows
            return pltpu.async_copy(flat_hbm.at[pl.ds(r0, crows)],
                                    bufs[ci % 2], sems[ci % 2])

        cp = _start(0)
        for ci in range(nch):
            cp.wait()
            if ci + 1 < nch:
                cp = _start(ci + 1)
            buf = bufs[ci % 2]

            def _rowloop(r, carry, buf=buf):
                def _scat(c, cc):
                    v = buf[r, pl.ds(c * 16, 16)]
                    plsc.addupdate_scatter(tab, [laneoff + v], ones)
                    return cc

                return lax.fori_loop(0, nvec_row, _scat, carry, unroll=4)

            lax.fori_loop(0, crows, _rowloop, 0)

        def _red(j, c):
            acc = tab[pl.ds(j * 16, 16)]
            for l in range(1, 16):
                acc = acc + tab[pl.ds(l * _TSTRIDE + j * 16, 16)]
            hist[pl.ds(j * 16, 16)] = acc
            return c

        lax.fori_loop(0, _NT16, _red, 0)
        pltpu.sync_copy(hist, out_hbm.at[wid])

    return _sc_body, crows


def _sc_hist(flat2d):
    rows = flat2d.shape[0]
    body, crows = _make_sc_body(rows)
    mesh = plsc.VectorSubcoreMesh(
        core_axis_name="c", subcore_axis_name="s", num_cores=_NCORE)
    return pl.kernel(
        body,
        out_type=jax.ShapeDtypeStruct((_NWORK, _TSTRIDE), jnp.float32),
        mesh=mesh,
        compiler_params=pltpu.CompilerParams(needs_layout_passes=False),
        scratch_types=[
            pltpu.VMEM((crows, _W), jnp.int32),
            pltpu.VMEM((crows, _W), jnp.int32),
            pltpu.VMEM((16 * _TSTRIDE,), jnp.float32),
            pltpu.VMEM((_TSTRIDE,), jnp.float32),
            pltpu.SemaphoreType.DMA,
            pltpu.SemaphoreType.DMA,
        ],
    )(flat2d)


def _iou_body(h_ref, out_ref):
    conf = jnp.sum(h_ref[...], axis=0, keepdims=True)
    b = lax.broadcasted_iota(jnp.int32, (_TSTRIDE, NCLS), 0)
    i = lax.broadcasted_iota(jnp.int32, (_TSTRIDE, NCLS), 1)
    valid = b < NCLS * NCLS
    mrow = ((b // NCLS == i) & valid).astype(jnp.float32)
    mcol = ((b % NCLS == i) & valid).astype(jnp.float32)
    mtp = (b == i * (NCLS + 1)).astype(jnp.float32)
    dn = (((1,), (0,)), ((), ()))
    hp = lax.Precision.HIGHEST
    row = lax.dot_general(conf, mrow, dn, precision=hp,
                          preferred_element_type=jnp.float32)
    col = lax.dot_general(conf, mcol, dn, precision=hp,
                          preferred_element_type=jnp.float32)
    tp = lax.dot_general(conf, mtp, dn, precision=hp,
                         preferred_element_type=jnp.float32)
    iou = (tp + _EPS) / (row + col - tp + _EPS)
    mask = (row > 0).astype(jnp.float32)
    num = jnp.sum(iou * mask, keepdims=True)
    den = jnp.maximum(jnp.sum(mask, keepdims=True), 1.0)
    out_ref[...] = num / den


def _iou_tc(hists):
    return pl.pallas_call(
        _iou_body,
        out_shape=jax.ShapeDtypeStruct((1, 1), jnp.float32),
    )(hists)


_SLICES = (384, 128)


def kernel(y_pr, y_gt):
    row0 = 0
    parts = []
    for srows in _SLICES:
        fl = _argmax_flat(y_pr, y_gt, row0, srows)
        parts.append(_sc_hist(fl))
        row0 += srows
    hists = jnp.concatenate(parts, axis=0)
    out = _iou_tc(hists)
    return out[0, 0]


# speedup vs baseline: 1.0322x; 1.0308x over previous
import jax
import jax.numpy as jnp
from jax import lax
from jax.experimental import pallas as pl
from jax.experimental.pallas import tpu as pltpu
from jax.experimental.pallas import tpu_sc as plsc

NCLS = 19
_EPS = 1e-07


_R = 128


def _tc_body(pr_ref, gt_ref, out_ref):
    def amax(x):
        m = x[0]
        idx = jnp.zeros(m.shape, jnp.int32)
        for k in range(1, NCLS):
            xk = x[k]
            b = xk > m
            m = jnp.where(b, xk, m)
            idx = jnp.where(b, k, idx)
        return idx

    out_ref[...] = amax(gt_ref[0]) * NCLS + amax(pr_ref[0])


def _argmax_flat(y_pr, y_gt, row0, nrows):
    n, k, h, w = y_pr.shape
    nrb = nrows // _R
    rb0 = row0 // _R
    return pl.pallas_call(
        _tc_body,
        grid=(n, nrb),
        in_specs=[
            pl.BlockSpec((1, k, _R, w), lambda i, j: (i, 0, j + rb0, 0)),
            pl.BlockSpec((1, k, _R, w), lambda i, j: (i, 0, j + rb0, 0)),
        ],
        out_specs=pl.BlockSpec((_R, w), lambda i, j: (i * nrb + j, 0)),
        out_shape=jax.ShapeDtypeStruct((n * nrows, w), jnp.int32),
    )(y_pr, y_gt)


_W = 512
_NCORE = 2
_NWORK = 16 * _NCORE
_TSTRIDE = 368
_NT16 = _TSTRIDE // 16


def _make_sc_body(rows):
    rpw = rows // _NWORK
    nch = max(1, (rpw + 31) // 32)
    crows = rpw // nch

    def _sc_body(flat_hbm, out_hbm, buf0, buf1, tab, hist, sem0, sem1):
        wid = lax.axis_index("s") * _NCORE + lax.axis_index("c")
        iota = lax.iota(jnp.int32, 16)
        ones = jnp.ones((16,), jnp.float32)
        zeros = jnp.zeros((16,), jnp.float32)
        laneoff = iota * _TSTRIDE

        def _zero(i, c):
            tab[pl.ds(i * 16, 16)] = zeros
            return c

        lax.fori_loop(0, 16 * _NT16, _zero, 0)

        bufs = (buf0, buf1)
        sems = (sem0, sem1)
        nvec_row = _W // 16

        def _start(ci):
            r0 = wid * rpw + ci * crows
            return pltpu.async_copy(flat_hbm.at[pl.ds(r0, crows)],
                                    bufs[ci % 2], sems[ci % 2])

        cp = _start(0)
        for ci in range(nch):
            cp.wait()
            if ci + 1 < nch:
                cp = _start(ci + 1)
            buf = bufs[ci % 2]

            def _rowloop(r, carry, buf=buf):
                def _scat(c, cc):
                    v = buf[r, pl.ds(c * 16, 16)]
                    plsc.addupdate_scatter(tab, [laneoff + v], ones)
                    return cc

                return lax.fori_loop(0, nvec_row, _scat, carry, unroll=8)

            lax.fori_loop(0, crows, _rowloop, 0)

        def _red(j, c):
            acc = tab[pl.ds(j * 16, 16)]
            for l in range(1, 16):
                acc = acc + tab[pl.ds(l * _TSTRIDE + j * 16, 16)]
            hist[pl.ds(j * 16, 16)] = acc
            return c

        lax.fori_loop(0, _NT16, _red, 0)
        pltpu.sync_copy(hist, out_hbm.at[wid])

    return _sc_body, crows


def _sc_hist(flat2d):
    rows = flat2d.shape[0]
    body, crows = _make_sc_body(rows)
    mesh = plsc.VectorSubcoreMesh(
        core_axis_name="c", subcore_axis_name="s", num_cores=_NCORE)
    return pl.kernel(
        body,
        out_type=jax.ShapeDtypeStruct((_NWORK, _TSTRIDE), jnp.float32),
        mesh=mesh,
        compiler_params=pltpu.CompilerParams(needs_layout_passes=False),
        scratch_types=[
            pltpu.VMEM((crows, _W), jnp.int32),
            pltpu.VMEM((crows, _W), jnp.int32),
            pltpu.VMEM((16 * _TSTRIDE,), jnp.float32),
            pltpu.VMEM((_TSTRIDE,), jnp.float32),
            pltpu.SemaphoreType.DMA,
            pltpu.SemaphoreType.DMA,
        ],
    )(flat2d)


def _iou_body(*refs):
    h_refs, out_ref = refs[:-1], refs[-1]
    conf = jnp.zeros((1, _TSTRIDE), jnp.float32)
    for h_ref in h_refs:
        conf = conf + jnp.sum(h_ref[...], axis=0, keepdims=True)
    b = lax.broadcasted_iota(jnp.int32, (_TSTRIDE, NCLS), 0)
    i = lax.broadcasted_iota(jnp.int32, (_TSTRIDE, NCLS), 1)
    valid = b < NCLS * NCLS
    mrow = ((b // NCLS == i) & valid).astype(jnp.float32)
    mcol = ((b % NCLS == i) & valid).astype(jnp.float32)
    mtp = (b == i * (NCLS + 1)).astype(jnp.float32)
    dn = (((1,), (0,)), ((), ()))
    hp = lax.Precision.HIGHEST
    row = lax.dot_general(conf, mrow, dn, precision=hp,
                          preferred_element_type=jnp.float32)
    col = lax.dot_general(conf, mcol, dn, precision=hp,
                          preferred_element_type=jnp.float32)
    tp = lax.dot_general(conf, mtp, dn, precision=hp,
                         preferred_element_type=jnp.float32)
    iou = (tp + _EPS) / (row + col - tp + _EPS)
    mask = (row > 0).astype(jnp.float32)
    num = jnp.sum(iou * mask, keepdims=True)
    den = jnp.maximum(jnp.sum(mask, keepdims=True), 1.0)
    out_ref[...] = num / den


def _iou_tc(parts):
    return pl.pallas_call(
        _iou_body,
        out_shape=jax.ShapeDtypeStruct((1, 1), jnp.float32),
    )(*parts)


_SLICES = (384, 128)


def kernel(y_pr, y_gt):
    row0 = 0
    parts = []
    for srows in _SLICES:
        fl = _argmax_flat(y_pr, y_gt, row0, srows)
        parts.append(_sc_hist(fl))
        row0 += srows
    out = _iou_tc(parts)
    return out[0, 0]
